# baseline (speedup 1.0000x reference)
import functools

import jax
import jax.numpy as jnp
from jax import lax
from jax.experimental import pallas as pl
from jax.experimental.pallas import tpu as pltpu
from jax.experimental.pallas import tpu_sc as plsc

N = 100000
F = 128
G = 1000
W = 64
NWIN = (G + W - 1) // W
GP = NWIN * W
B1 = 4000
NT1 = N // B1
CH = 800
NCH = N // CH
NWORK = 32
EPS = 1e-5


def _h_body(x_ref, seg_ref, w1_ref, b1_ref, w2_ref, b2_ref,
            h_ref, q_ref, cnt_ref, qacc_ref, cacc_ref):
    i = pl.program_id(0)

    @pl.when(i == 0)
    def _init():
        qacc_ref[...] = jnp.zeros_like(qacc_ref)
        cacc_ref[...] = jnp.zeros_like(cacc_ref)

    x = x_ref[...]
    l1 = lax.dot_general(x, w1_ref[...], (((1,), (1,)), ((), ())),
                         preferred_element_type=jnp.float32) + b1_ref[...]
    l2 = lax.dot_general(x, w2_ref[...], (((1,), (1,)), ((), ())),
                         preferred_element_type=jnp.float32) + b2_ref[...]
    h = l1 * l2
    h_ref[...] = h
    qacc_ref[...] += jnp.sum(h * h, axis=0, keepdims=True)

    seg = seg_ref[0]
    lo = jnp.min(seg)
    hi = jnp.max(seg)
    iota_w = lax.broadcasted_iota(jnp.int32, (W, B1), 0)
    for k in range(NWIN):
        @pl.when((hi >= k * W) & (lo < (k + 1) * W))
        def _window(k=k):
            ohf = (iota_w == (seg - k * W)).astype(jnp.float32)
            cnt = jnp.sum(ohf, axis=1, keepdims=True)
            cacc_ref[k * W:(k + 1) * W, :] += jnp.broadcast_to(cnt, (W, F))

    @pl.when(i == NT1 - 1)
    def _fin():
        q_ref[...] = qacc_ref[...]
        cnt_ref[...] = cacc_ref[...]


def _h_stage(x, seg3, W1, b1, W2, b2):
    return pl.pallas_call(
        _h_body,
        grid=(NT1,),
        in_specs=[
            pl.BlockSpec((B1, F), lambda i: (i, 0)),
            pl.BlockSpec((1, 1, B1), lambda i: (i, 0, 0)),
            pl.BlockSpec((F, F), lambda i: (0, 0)),
            pl.BlockSpec((1, F), lambda i: (0, 0)),
            pl.BlockSpec((F, F), lambda i: (0, 0)),
            pl.BlockSpec((1, F), lambda i: (0, 0)),
        ],
        out_specs=[
            pl.BlockSpec((B1, F), lambda i: (i, 0)),
            pl.BlockSpec((1, F), lambda i: (0, 0)),
            pl.BlockSpec((GP, F), lambda i: (0, 0)),
        ],
        out_shape=[
            jax.ShapeDtypeStruct((N, F), jnp.float32),
            jax.ShapeDtypeStruct((1, F), jnp.float32),
            jax.ShapeDtypeStruct((GP, F), jnp.float32),
        ],
        scratch_shapes=[
            pltpu.VMEM((1, F), jnp.float32),
            pltpu.VMEM((GP, F), jnp.float32),
        ],
        compiler_params=pltpu.CompilerParams(
            dimension_semantics=("arbitrary",),
        ),
    )(x, seg3, W1, b1, W2, b2)


def _sc_segsum(h, seg, zeros_p):
    mesh = plsc.VectorSubcoreMesh(core_axis_name="c", subcore_axis_name="s")

    @functools.partial(
        pl.kernel,
        mesh=mesh,
        out_type=jax.ShapeDtypeStruct((2, GP, F), jnp.float32),
        scratch_types=[
            pltpu.VMEM((CH, F), jnp.float32),
            pltpu.VMEM((CH,), jnp.int32),
            pltpu.VMEM_SHARED((GP, F), jnp.float32),
        ],
    )
    def k(h_hbm, seg_hbm, zp_hbm, p_out, hbuf, idxbuf, accp):
        c = lax.axis_index("c")
        s = lax.axis_index("s")
        wid = s * 2 + c

        pltpu.sync_copy(zp_hbm.at[pl.ds(s * 64, 64)],
                        accp.at[pl.ds(s * 64, 64)])
        plsc.subcore_barrier()

        def body(j, carry):
            cid = wid + NWORK * j

            @pl.when(cid < NCH)
            def _chunk():
                pltpu.sync_copy(seg_hbm.at[pl.ds(cid * CH, CH)], idxbuf)
                pltpu.sync_copy(h_hbm.at[pl.ds(cid * CH, CH)], hbuf)
                pltpu.sync_copy(hbuf, accp.at[idxbuf], add=True)
            return carry

        jax.lax.fori_loop(0, (NCH + NWORK - 1) // NWORK, body, 0)
        plsc.subcore_barrier()

        @pl.when(s == 0)
        def _readout():
            pltpu.sync_copy(accp, p_out.at[c])

    return k(h, seg, zeros_p)


def _fix_body(p_ref, c_ref, q_ref, g_ref, be_ref, out_ref):
    p = p_ref[0] + p_ref[1]
    inv_n = 1.0 / N
    mean = jnp.sum(p, axis=0, keepdims=True) * inv_n
    var = jnp.maximum(q_ref[...] * inv_n - mean * mean, 0.0)
    scale = g_ref[...] * lax.rsqrt(var + EPS)
    shift = be_ref[...] - mean * scale
    out_ref[...] = p[:G, :] * scale + c_ref[:G, :] * shift


def _fix_stage(p, cnt, q, gamma, beta):
    return pl.pallas_call(
        _fix_body,
        out_shape=jax.ShapeDtypeStruct((G, F), jnp.float32),
    )(p, cnt, q, gamma, beta)


@jax.jit
def _run(x, seg, seg3, W1, b1, W2, b2, gamma, beta):
    h, q, cnt = _h_stage(x, seg3, W1, b1, W2, b2)
    zeros_p = jnp.zeros((GP, F), jnp.float32)
    p = _sc_segsum(h, seg, zeros_p)
    return _fix_stage(p, cnt, q, gamma, beta)


def kernel(input, graph_indices, node_counts, W1, b1, W2, b2, gamma, beta):
    del node_counts
    seg = graph_indices.astype(jnp.int32)
    seg3 = seg.reshape(NT1, 1, B1)
    return _run(input, seg, seg3, W1, b1.reshape(1, F), W2, b2.reshape(1, F),
                gamma.reshape(1, F), beta.reshape(1, F))


# speedup vs baseline: 1.0277x; 1.0277x over previous
import functools

import jax
import jax.numpy as jnp
from jax import lax
from jax.experimental import pallas as pl
from jax.experimental.pallas import tpu as pltpu
from jax.experimental.pallas import tpu_sc as plsc

N = 100000
F = 128
G = 1000
W = 64
NWIN = (G + W - 1) // W
GP = NWIN * W
B1 = 4000
NT1 = N // B1
CH = 400
NCH = N // CH
NWORK = 32
EPS = 1e-5


def _h_body(x_ref, seg_ref, w1_ref, b1_ref, w2_ref, b2_ref,
            h_ref, q_ref, cnt_ref, qacc_ref, cacc_ref):
    i = pl.program_id(0)

    @pl.when(i == 0)
    def _init():
        qacc_ref[...] = jnp.zeros_like(qacc_ref)
        cacc_ref[...] = jnp.zeros_like(cacc_ref)

    x = x_ref[...]
    l1 = lax.dot_general(x, w1_ref[...], (((1,), (1,)), ((), ())),
                         preferred_element_type=jnp.float32) + b1_ref[...]
    l2 = lax.dot_general(x, w2_ref[...], (((1,), (1,)), ((), ())),
                         preferred_element_type=jnp.float32) + b2_ref[...]
    h = l1 * l2
    h_ref[...] = h
    qacc_ref[...] += jnp.sum(h * h, axis=0, keepdims=True)

    seg = seg_ref[0]
    lo = jnp.min(seg)
    hi = jnp.max(seg)
    iota_w = lax.broadcasted_iota(jnp.int32, (W, B1), 0)
    for k in range(NWIN):
        @pl.when((hi >= k * W) & (lo < (k + 1) * W))
        def _window(k=k):
            ohf = (iota_w == (seg - k * W)).astype(jnp.float32)
            cnt = jnp.sum(ohf, axis=1, keepdims=True)
            cacc_ref[k * W:(k + 1) * W, :] += jnp.broadcast_to(cnt, (W, F))

    @pl.when(i == NT1 - 1)
    def _fin():
        q_ref[...] = qacc_ref[...]
        cnt_ref[...] = cacc_ref[...]


def _h_stage(x, seg3, W1, b1, W2, b2):
    return pl.pallas_call(
        _h_body,
        grid=(NT1,),
        in_specs=[
            pl.BlockSpec((B1, F), lambda i: (i, 0)),
            pl.BlockSpec((1, 1, B1), lambda i: (i, 0, 0)),
            pl.BlockSpec((F, F), lambda i: (0, 0)),
            pl.BlockSpec((1, F), lambda i: (0, 0)),
            pl.BlockSpec((F, F), lambda i: (0, 0)),
            pl.BlockSpec((1, F), lambda i: (0, 0)),
        ],
        out_specs=[
            pl.BlockSpec((B1, F), lambda i: (i, 0)),
            pl.BlockSpec((1, F), lambda i: (0, 0)),
            pl.BlockSpec((GP, F), lambda i: (0, 0)),
        ],
        out_shape=[
            jax.ShapeDtypeStruct((N, F), jnp.float32),
            jax.ShapeDtypeStruct((1, F), jnp.float32),
            jax.ShapeDtypeStruct((GP, F), jnp.float32),
        ],
        scratch_shapes=[
            pltpu.VMEM((1, F), jnp.float32),
            pltpu.VMEM((GP, F), jnp.float32),
        ],
        compiler_params=pltpu.CompilerParams(
            dimension_semantics=("arbitrary",),
        ),
    )(x, seg3, W1, b1, W2, b2)


def _sc_segsum(h, seg, zeros_p):
    mesh = plsc.VectorSubcoreMesh(core_axis_name="c", subcore_axis_name="s")

    @functools.partial(
        pl.kernel,
        mesh=mesh,
        out_type=jax.ShapeDtypeStruct((2, GP, F), jnp.float32),
        scratch_types=[
            pltpu.VMEM((CH, F), jnp.float32),
            pltpu.VMEM((CH, F), jnp.float32),
            pltpu.VMEM((CH,), jnp.int32),
            pltpu.VMEM((CH,), jnp.int32),
            pltpu.VMEM_SHARED((GP, F), jnp.float32),
            pltpu.SemaphoreType.DMA,
            pltpu.SemaphoreType.DMA,
        ],
    )
    def k(h_hbm, seg_hbm, zp_hbm, p_out, hbuf0, hbuf1, idxbuf0, idxbuf1,
          accp, sem0, sem1):
        c = lax.axis_index("c")
        s = lax.axis_index("s")
        wid = s * 2 + c
        sems = (sem0, sem1)
        hbufs = (hbuf0, hbuf1)
        idxbufs = (idxbuf0, idxbuf1)

        pltpu.sync_copy(zp_hbm.at[pl.ds(s * 64, 64)],
                        accp.at[pl.ds(s * 64, 64)])
        plsc.subcore_barrier()

        nloc = (NCH + NWORK - 1) // NWORK

        def start_load(j, slot):
            cid = wid + NWORK * j

            @pl.when(cid < NCH)
            def _go():
                pltpu.async_copy(seg_hbm.at[pl.ds(cid * CH, CH)],
                                 idxbufs[slot], sems[slot])
                pltpu.async_copy(h_hbm.at[pl.ds(cid * CH, CH)],
                                 hbufs[slot], sems[slot])

        start_load(0, 0)
        for j in range(nloc):
            slot = j % 2
            if j + 1 < nloc:
                start_load(j + 1, (j + 1) % 2)
            cid = wid + NWORK * j

            @pl.when(cid < NCH)
            def _consume(slot=slot):
                pltpu.make_async_copy(seg_hbm.at[pl.ds(0, CH)],
                                      idxbufs[slot], sems[slot]).wait()
                pltpu.make_async_copy(h_hbm.at[pl.ds(0, CH)],
                                      hbufs[slot], sems[slot]).wait()
                pltpu.sync_copy(hbufs[slot], accp.at[idxbufs[slot]],
                                add=True)

        plsc.subcore_barrier()

        @pl.when(s == 0)
        def _readout():
            pltpu.sync_copy(accp, p_out.at[c])

    return k(h, seg, zeros_p)


def _fix_body(p_ref, c_ref, q_ref, g_ref, be_ref, out_ref):
    p = p_ref[0] + p_ref[1]
    inv_n = 1.0 / N
    mean = jnp.sum(p, axis=0, keepdims=True) * inv_n
    var = jnp.maximum(q_ref[...] * inv_n - mean * mean, 0.0)
    scale = g_ref[...] * lax.rsqrt(var + EPS)
    shift = be_ref[...] - mean * scale
    out_ref[...] = p[:G, :] * scale + c_ref[:G, :] * shift


def _fix_stage(p, cnt, q, gamma, beta):
    return pl.pallas_call(
        _fix_body,
        out_shape=jax.ShapeDtypeStruct((G, F), jnp.float32),
    )(p, cnt, q, gamma, beta)


@jax.jit
def _run(x, seg, seg3, W1, b1, W2, b2, gamma, beta):
    h, q, cnt = _h_stage(x, seg3, W1, b1, W2, b2)
    zeros_p = jnp.zeros((GP, F), jnp.float32)
    p = _sc_segsum(h, seg, zeros_p)
    return _fix_stage(p, cnt, q, gamma, beta)


def kernel(input, graph_indices, node_counts, W1, b1, W2, b2, gamma, beta):
    del node_counts
    seg = graph_indices.astype(jnp.int32)
    seg3 = seg.reshape(NT1, 1, B1)
    return _run(input, seg, seg3, W1, b1.reshape(1, F), W2, b2.reshape(1, F),
                gamma.reshape(1, F), beta.reshape(1, F))
